# baseline (speedup 1.0000x reference)
import functools

import jax
import jax.numpy as jnp
from jax import lax
from jax.experimental import pallas as pl
from jax.experimental.pallas import tpu as pltpu
from jax.experimental.pallas import tpu_sc as plsc

_NC = 2
_NS = 16
_NW = _NC * _NS
_NP = 10240
_K0 = 8
_K1 = 2
_D = 128


def _sc_mesh():
    return plsc.VectorSubcoreMesh(core_axis_name="c", subcore_axis_name="s")


def _scale_rows(rows, cv, cv_row, nrows):
    rowv = jnp.full((16,), cv_row, jnp.int32)

    def srow(r, _):
        s = plsc.load_gather(cv, [rowv, jnp.full((16,), r, jnp.int32)])
        for cc in range(_D // 16):
            rows[r, pl.ds(cc * 16, 16)] = rows[r, pl.ds(cc * 16, 16)] * s
        return 0

    lax.fori_loop(0, nrows, srow, 0, unroll=2)


def _add_rows(a, b, nrows):
    def arow(r, _):
        for cc in range(_D // 16):
            a[r, pl.ds(cc * 16, 16)] = (a[r, pl.ds(cc * 16, 16)]
                                        + b[r, pl.ds(cc * 16, 16)])
        return 0

    lax.fori_loop(0, nrows, arow, 0, unroll=2)


def _zero_rows(buf, nrows):
    z = jnp.zeros((16,), jnp.float32)

    def zrow(r, _):
        for cc in range(_D // 16):
            buf[r, pl.ds(cc * 16, 16)] = z
        return 0

    lax.fori_loop(0, nrows, zrow, 0, unroll=2)


def _sc_seg(table, src2, dst2, coef2):
    R = src2.shape[0]
    CR = 8
    K = R // (CR * _NW)
    NR = _NP // _NS

    @functools.partial(
        pl.kernel,
        out_type=jax.ShapeDtypeStruct((2 * _NP, _D), jnp.float32),
        mesh=_sc_mesh(),
        compiler_params=pltpu.CompilerParams(needs_layout_passes=False),
        scratch_types=[
            pltpu.VMEM((CR, 128), jnp.int32),
            pltpu.VMEM((CR, 128), jnp.int32),
            pltpu.VMEM((CR, 128), jnp.float32),
            pltpu.VMEM((128, _D), jnp.float32),
            pltpu.VMEM((128, _D), jnp.float32),
            pltpu.VMEM_SHARED((_NP, _D), jnp.float32),
            pltpu.SemaphoreType.DMA,
        ],
    )
    def k(t_hbm, src_hbm, dst_hbm, coef_hbm, out_hbm,
          isrc, idst, cv, r0, r1, acc, sem):
        cid = lax.axis_index("c")
        sid = lax.axis_index("s")
        kc = jnp.where(cid == 0, _K0 * K // 5, _K1 * K // 5)
        gbase = cid * _NS * (_K0 * K // 5) + sid * kc
        rbuf = (r0, r1)
        _zero_rows(r0, 128)
        for t in range(NR // 128):
            pltpu.sync_copy(r0, acc.at[pl.ds(sid * NR + t * 128, 128)])
        plsc.subcore_barrier()

        def chunk(j, carry):
            g = gbase + j
            ib = g * CR
            pltpu.sync_copy(src_hbm.at[pl.ds(ib, CR)], isrc)
            pltpu.sync_copy(dst_hbm.at[pl.ds(ib, CR)], idst)
            pltpu.sync_copy(coef_hbm.at[pl.ds(ib, CR)], cv)
            gcp = pltpu.async_copy(t_hbm.at[isrc.at[0]], rbuf[0], sem)
            for p in range(CR):
                ncp = None
                if p + 1 < CR:
                    ncp = pltpu.async_copy(t_hbm.at[isrc.at[p + 1]],
                                           rbuf[(p + 1) % 2], sem)
                gcp.wait()
                _scale_rows(rbuf[p % 2], cv, p, 128)
                pltpu.sync_copy(rbuf[p % 2], acc.at[idst.at[p]], add=True)
                gcp = ncp
            return 0

        lax.fori_loop(0, kc, chunk, 0)
        plsc.subcore_barrier()
        pltpu.sync_copy(acc.at[pl.ds(sid * NR, NR)],
                        out_hbm.at[pl.ds(cid * _NP + sid * NR, NR)])

    return k(table, src2, dst2, coef2)


def _sc_pair(A, B, src2, dst2):
    R = src2.shape[0]
    CR = 8
    C = CR * 128
    K = R // (CR * _NW)
    E2P = R * 128

    @functools.partial(
        pl.kernel,
        out_type=jax.ShapeDtypeStruct((E2P, _D), jnp.float32),
        mesh=_sc_mesh(),
        compiler_params=pltpu.CompilerParams(needs_layout_passes=False),
        scratch_types=[
            pltpu.VMEM((CR, 128), jnp.int32),
            pltpu.VMEM((CR, 128), jnp.int32),
            pltpu.VMEM((128, _D), jnp.float32),
            pltpu.VMEM((128, _D), jnp.float32),
            pltpu.VMEM((128, _D), jnp.float32),
            pltpu.VMEM((128, _D), jnp.float32),
            pltpu.VMEM((128, _D), jnp.float32),
            pltpu.SemaphoreType.DMA,
            pltpu.SemaphoreType.DMA,
        ],
    )
    def k(a_hbm, b_hbm, src_hbm, dst_hbm, s_hbm,
          isrc, idst, a0, a1, a2, b0, b1, semg, semw):
        cid = lax.axis_index("c")
        sid = lax.axis_index("s")
        kc = jnp.where(cid == 0, _K0 * K // 5, _K1 * K // 5)
        gbase = cid * _NS * (_K0 * K // 5) + sid * kc
        abuf = (a0, a1, a2)
        bbuf = (b0, b1)

        def chunk(j, carry):
            g = gbase + j
            ib = g * CR
            eb = g * C
            pltpu.sync_copy(src_hbm.at[pl.ds(ib, CR)], isrc)
            pltpu.sync_copy(dst_hbm.at[pl.ds(ib, CR)], idst)
            gcp = [pltpu.async_copy(a_hbm.at[isrc.at[0]], abuf[0], semg),
                   pltpu.async_copy(b_hbm.at[idst.at[0]], bbuf[0], semg)]
            wcp = [None] * CR
            for p in range(CR):
                if p >= 2:
                    wcp[p - 2].wait()
                ncp = None
                if p + 1 < CR:
                    ncp = [pltpu.async_copy(a_hbm.at[isrc.at[p + 1]],
                                            abuf[(p + 1) % 3], semg),
                           pltpu.async_copy(b_hbm.at[idst.at[p + 1]],
                                            bbuf[(p + 1) % 2], semg)]
                for cp in gcp:
                    cp.wait()
                _add_rows(abuf[p % 3], bbuf[p % 2], 128)
                wcp[p] = pltpu.async_copy(abuf[p % 3],
                                          s_hbm.at[pl.ds(eb + p * 128, 128)],
                                          semw)
                gcp = ncp
            wcp[CR - 2].wait()
            wcp[CR - 1].wait()
            return 0

        lax.fori_loop(0, kc, chunk, 0)

    return k(A, B, src2, dst2)


def _tc0(x, Wcat, angp2, ewp2):
    N = x.shape[0]
    R = angp2.shape[0]
    BN = N // 10
    BR = R // 10

    def body(x_ref, w_ref, a_ref, e_ref, xy_ref, coef_ref):
        xy_ref[...] = jnp.dot(x_ref[...], w_ref[...],
                              preferred_element_type=jnp.float32)
        coef_ref[...] = e_ref[...] * jnp.cos(a_ref[...])

    return pl.pallas_call(
        body,
        grid=(10,),
        in_specs=[
            pl.BlockSpec((BN, 128), lambda i: (i, 0)),
            pl.BlockSpec((128, 128), lambda i: (0, 0)),
            pl.BlockSpec((BR, 128), lambda i: (i, 0)),
            pl.BlockSpec((BR, 128), lambda i: (i, 0)),
        ],
        out_specs=[
            pl.BlockSpec((BN, 128), lambda i: (i, 0)),
            pl.BlockSpec((BR, 128), lambda i: (i, 0)),
        ],
        out_shape=[
            jax.ShapeDtypeStruct((N, 128), jnp.float32),
            jax.ShapeDtypeStruct((R, 128), jnp.float32),
        ],
    )(x, Wcat, angp2, ewp2)


def _tc1(xy, p3, b1, Wbig):
    N = xy.shape[0]
    BN = N // 10

    def body(xy_ref, p0_ref, p1_ref, b_ref, w_ref,
             xf_ref, u_ref, v_ref, y2_ref):
        xs = xy_ref[...][:, :64]
        a = p0_ref[...][0, :, 64:] + p1_ref[...][0, :, 64:]
        x1 = jnp.maximum(xs + a + b_ref[...], 0.0)
        big = jnp.dot(x1, w_ref[...], preferred_element_type=jnp.float32)
        xf_ref[...] = big[:, :128]
        u_ref[...] = big[:, 128:256]
        v_ref[...] = big[:, 256:384]
        y2_ref[...] = big[:, 384:]

    blk = lambda i: (i, 0)
    full = lambda i: (0, 0)
    return pl.pallas_call(
        body,
        grid=(10,),
        in_specs=[
            pl.BlockSpec((BN, 128), blk),
            pl.BlockSpec((1, BN, 128), lambda i: (0, i, 0)),
            pl.BlockSpec((1, BN, 128), lambda i: (1, i, 0)),
            pl.BlockSpec((1, 64), full),
            pl.BlockSpec((64, 512), full),
        ],
        out_specs=[pl.BlockSpec((BN, 128), blk)] * 4,
        out_shape=[jax.ShapeDtypeStruct((N, 128), jnp.float32)] * 4,
    )(xy, p3, p3, b1, Wbig)


def _tc2(xf2, p3, b2, Wpq):
    N = xf2.shape[0]
    BN = N // 10

    def body(xf_ref, p0_ref, p1_ref, b_ref, w_ref, p_ref, q_ref):
        x2 = jnp.maximum(xf_ref[...] + p0_ref[...][0] + p1_ref[...][0]
                         + b_ref[...], 0.0)
        pq = jnp.dot(x2, w_ref[...], preferred_element_type=jnp.float32)
        p_ref[...] = pq[:, :128]
        q_ref[...] = pq[:, 128:]

    blk = lambda i: (i, 0)
    full = lambda i: (0, 0)
    return pl.pallas_call(
        body,
        grid=(10,),
        in_specs=[
            pl.BlockSpec((BN, 128), blk),
            pl.BlockSpec((1, BN, 128), lambda i: (0, i, 0)),
            pl.BlockSpec((1, BN, 128), lambda i: (1, i, 0)),
            pl.BlockSpec((1, 128), full),
            pl.BlockSpec((128, 256), full),
        ],
        out_specs=[pl.BlockSpec((BN, 128), blk)] * 2,
        out_shape=[jax.ShapeDtypeStruct((N, 128), jnp.float32)] * 2,
    )(xf2, p3, p3, b2, Wpq)


def _tc3(E, Ssum, ef, ewc, be1, We2c, be2, Wo1a, Wo1b, wo1c, bo1, Wo2, bo2):
    E2 = ef.shape[0]
    BE = 640
    G = E2 // BE

    def body(e_ref, s_ref, ef_ref, ew_ref,
             b1_ref, w2c_ref, b2_ref,
             wo1a_ref, wo1b_ref, wo1c_ref, bo1_ref, wo2_ref, bo2_ref,
             out_ref):
        ew = ew_ref[...]
        e1 = jnp.maximum(ew * e_ref[...] + b1_ref[...], 0.0)
        e2 = jnp.maximum(
            ew * s_ref[...]
            + jnp.dot(e1, w2c_ref[...], preferred_element_type=jnp.float32)
            + b2_ref[...], 0.0)
        h = (jnp.dot(e2, wo1a_ref[...], preferred_element_type=jnp.float32)
             + jnp.dot(ef_ref[...], wo1b_ref[...], preferred_element_type=jnp.float32)
             + ew * wo1c_ref[...]
             + bo1_ref[...])
        logits = jnp.dot(h, wo2_ref[...], preferred_element_type=jnp.float32) + bo2_ref[...]
        m = jnp.max(logits, axis=-1, keepdims=True)
        ex = jnp.exp(logits - m)
        out_ref[...] = ex / jnp.sum(ex, axis=-1, keepdims=True)

    full = lambda i: (0, 0)
    blk = lambda i: (i, 0)
    return pl.pallas_call(
        body,
        grid=(G,),
        in_specs=[
            pl.BlockSpec((BE, 128), blk),
            pl.BlockSpec((BE, 128), blk),
            pl.BlockSpec((BE, 16), blk),
            pl.BlockSpec((BE, 1), blk),
            pl.BlockSpec((1, 128), full),
            pl.BlockSpec((128, 128), full),
            pl.BlockSpec((1, 128), full),
            pl.BlockSpec((128, 256), full),
            pl.BlockSpec((16, 256), full),
            pl.BlockSpec((1, 256), full),
            pl.BlockSpec((1, 256), full),
            pl.BlockSpec((256, 2), full),
            pl.BlockSpec((1, 2), full),
        ],
        out_specs=pl.BlockSpec((BE, 2), blk),
        out_shape=jax.ShapeDtypeStruct((E2, 2), jnp.float32),
    )(E, Ssum, ef, ewc, be1, We2c, be2, Wo1a, Wo1b, wo1c, bo1, Wo2, bo2)


def kernel(node_features, edge_features_1d, edge_index, angles, edge_weights,
           W_self1, W_nbr1, b_n1, We1, be1,
           W_self2, W_nbr2, b_n2, We2, be2,
           W_o1, b_o1, W_o2, b_o2):
    N = node_features.shape[0]
    E2 = angles.shape[0]
    CHUNK = 1024 * _NW
    E2P = ((E2 + CHUNK - 1) // CHUNK) * CHUNK
    pad = E2P - E2
    R = E2P // 128

    src = edge_index[0]
    dst = edge_index[1]
    ewf = jnp.concatenate([edge_weights, edge_weights])
    src2 = jnp.pad(src, (0, pad)).reshape(R, 128)
    dst2 = jnp.pad(dst, (0, pad)).reshape(R, 128)
    angp2 = jnp.pad(angles, (0, pad)).reshape(R, 128)
    ewp2 = jnp.pad(ewf, (0, pad)).reshape(R, 128)

    Wcat = jnp.concatenate([W_self1, W_nbr1], axis=1)
    Wbig = jnp.concatenate([W_self2, We1[:64], We1[64:], W_nbr2], axis=1)

    xy, coef2 = _tc0(node_features, Wcat, angp2, ewp2)
    agg1 = _sc_seg(xy, src2, dst2, coef2)
    xf2, U, V, Y2 = _tc1(xy, agg1.reshape(2, _NP, _D), b_n1.reshape(1, 64), Wbig)
    E = _sc_pair(U, V, src2, dst2)
    agg2 = _sc_seg(Y2, src2, dst2, coef2)
    Wpq = jnp.concatenate([We2[:128], We2[128:256]], axis=1)
    P, Q = _tc2(xf2, agg2.reshape(2, _NP, _D), b_n2.reshape(1, 128), Wpq)
    Ssum = _sc_pair(P, Q, src2, dst2)
    out = _tc3(E, Ssum, edge_features_1d, ewf.reshape(E2, 1),
               be1.reshape(1, 128),
               We2[256:], be2.reshape(1, 128),
               W_o1[:128], W_o1[128:144], W_o1[144:145], b_o1.reshape(1, 256),
               W_o2, b_o2.reshape(1, 2))
    return out


# speedup vs baseline: 1.0975x; 1.0975x over previous
import functools

import jax
import jax.numpy as jnp
from jax import lax
from jax.experimental import pallas as pl
from jax.experimental.pallas import tpu as pltpu
from jax.experimental.pallas import tpu_sc as plsc

_NC = 2
_NS = 16
_NW = _NC * _NS
_NP = 10240
_K0 = 7
_K1 = 3
_D = 128


def _sc_mesh():
    return plsc.VectorSubcoreMesh(core_axis_name="c", subcore_axis_name="s")


def _scale_rows(rows, cv, cv_row, nrows):
    rowv = jnp.full((16,), cv_row, jnp.int32)

    def srow(r, _):
        s = plsc.load_gather(cv, [rowv, jnp.full((16,), r, jnp.int32)])
        for cc in range(_D // 16):
            rows[r, pl.ds(cc * 16, 16)] = rows[r, pl.ds(cc * 16, 16)] * s
        return 0

    lax.fori_loop(0, nrows, srow, 0, unroll=2)


def _add_rows(a, b, nrows):
    def arow(r, _):
        for cc in range(_D // 16):
            a[r, pl.ds(cc * 16, 16)] = (a[r, pl.ds(cc * 16, 16)]
                                        + b[r, pl.ds(cc * 16, 16)])
        return 0

    lax.fori_loop(0, nrows, arow, 0, unroll=2)


def _zero_rows(buf, nrows):
    z = jnp.zeros((16,), jnp.float32)

    def zrow(r, _):
        for cc in range(_D // 16):
            buf[r, pl.ds(cc * 16, 16)] = z
        return 0

    lax.fori_loop(0, nrows, zrow, 0, unroll=2)


def _sc_seg(table, src2, dst2, coef2):
    R = src2.shape[0]
    CR = 8
    K = R // (CR * _NW)
    NR = _NP // _NS

    @functools.partial(
        pl.kernel,
        out_type=jax.ShapeDtypeStruct((2 * _NP, _D), jnp.float32),
        mesh=_sc_mesh(),
        compiler_params=pltpu.CompilerParams(needs_layout_passes=False),
        scratch_types=[
            pltpu.VMEM((CR, 128), jnp.int32),
            pltpu.VMEM((CR, 128), jnp.int32),
            pltpu.VMEM((CR, 128), jnp.float32),
            pltpu.VMEM((128, _D), jnp.float32),
            pltpu.VMEM((128, _D), jnp.float32),
            pltpu.VMEM_SHARED((_NP, _D), jnp.float32),
            pltpu.SemaphoreType.DMA,
        ],
    )
    def k(t_hbm, src_hbm, dst_hbm, coef_hbm, out_hbm,
          isrc, idst, cv, r0, r1, acc, sem):
        cid = lax.axis_index("c")
        sid = lax.axis_index("s")
        kc = jnp.where(cid == 0, _K0 * K // 5, _K1 * K // 5)
        gbase = cid * _NS * (_K0 * K // 5) + sid * kc
        rbuf = (r0, r1)
        _zero_rows(r0, 128)
        for t in range(NR // 128):
            pltpu.sync_copy(r0, acc.at[pl.ds(sid * NR + t * 128, 128)])
        plsc.subcore_barrier()

        def chunk(j, carry):
            g = gbase + j
            ib = g * CR
            pltpu.sync_copy(src_hbm.at[pl.ds(ib, CR)], isrc)
            pltpu.sync_copy(dst_hbm.at[pl.ds(ib, CR)], idst)
            pltpu.sync_copy(coef_hbm.at[pl.ds(ib, CR)], cv)
            gcp = pltpu.async_copy(t_hbm.at[isrc.at[0]], rbuf[0], sem)
            for p in range(CR):
                ncp = None
                if p + 1 < CR:
                    ncp = pltpu.async_copy(t_hbm.at[isrc.at[p + 1]],
                                           rbuf[(p + 1) % 2], sem)
                gcp.wait()
                _scale_rows(rbuf[p % 2], cv, p, 128)
                pltpu.sync_copy(rbuf[p % 2], acc.at[idst.at[p]], add=True)
                gcp = ncp
            return 0

        lax.fori_loop(0, kc, chunk, 0)
        plsc.subcore_barrier()
        pltpu.sync_copy(acc.at[pl.ds(sid * NR, NR)],
                        out_hbm.at[pl.ds(cid * _NP + sid * NR, NR)])

    return k(table, src2, dst2, coef2)


def _sc_pair(A, B, src2, dst2):
    R = src2.shape[0]
    CR = 8
    C = CR * 128
    K = R // (CR * _NW)
    E2P = R * 128

    @functools.partial(
        pl.kernel,
        out_type=jax.ShapeDtypeStruct((E2P, _D), jnp.float32),
        mesh=_sc_mesh(),
        compiler_params=pltpu.CompilerParams(needs_layout_passes=False),
        scratch_types=[
            pltpu.VMEM((CR, 128), jnp.int32),
            pltpu.VMEM((CR, 128), jnp.int32),
            pltpu.VMEM((128, _D), jnp.float32),
            pltpu.VMEM((128, _D), jnp.float32),
            pltpu.VMEM((128, _D), jnp.float32),
            pltpu.VMEM((128, _D), jnp.float32),
            pltpu.VMEM((128, _D), jnp.float32),
            pltpu.SemaphoreType.DMA,
            pltpu.SemaphoreType.DMA,
        ],
    )
    def k(a_hbm, b_hbm, src_hbm, dst_hbm, s_hbm,
          isrc, idst, a0, a1, a2, b0, b1, semg, semw):
        cid = lax.axis_index("c")
        sid = lax.axis_index("s")
        kc = jnp.where(cid == 0, _K0 * K // 5, _K1 * K // 5)
        gbase = cid * _NS * (_K0 * K // 5) + sid * kc
        abuf = (a0, a1, a2)
        bbuf = (b0, b1)

        def chunk(j, carry):
            g = gbase + j
            ib = g * CR
            eb = g * C
            pltpu.sync_copy(src_hbm.at[pl.ds(ib, CR)], isrc)
            pltpu.sync_copy(dst_hbm.at[pl.ds(ib, CR)], idst)
            gcp = [pltpu.async_copy(a_hbm.at[isrc.at[0]], abuf[0], semg),
                   pltpu.async_copy(b_hbm.at[idst.at[0]], bbuf[0], semg)]
            wcp = [None] * CR
            for p in range(CR):
                if p >= 2:
                    wcp[p - 2].wait()
                ncp = None
                if p + 1 < CR:
                    ncp = [pltpu.async_copy(a_hbm.at[isrc.at[p + 1]],
                                            abuf[(p + 1) % 3], semg),
                           pltpu.async_copy(b_hbm.at[idst.at[p + 1]],
                                            bbuf[(p + 1) % 2], semg)]
                for cp in gcp:
                    cp.wait()
                _add_rows(abuf[p % 3], bbuf[p % 2], 128)
                wcp[p] = pltpu.async_copy(abuf[p % 3],
                                          s_hbm.at[pl.ds(eb + p * 128, 128)],
                                          semw)
                gcp = ncp
            wcp[CR - 2].wait()
            wcp[CR - 1].wait()
            return 0

        lax.fori_loop(0, kc, chunk, 0)

    return k(A, B, src2, dst2)


def _tc0(x, Wcat, angp2, ewp2):
    N = x.shape[0]
    R = angp2.shape[0]
    BN = N // 10
    BR = R // 10

    def body(x_ref, w_ref, a_ref, e_ref, xy_ref, coef_ref):
        xy_ref[...] = jnp.dot(x_ref[...], w_ref[...],
                              preferred_element_type=jnp.float32)
        coef_ref[...] = e_ref[...] * jnp.cos(a_ref[...])

    return pl.pallas_call(
        body,
        grid=(10,),
        in_specs=[
            pl.BlockSpec((BN, 128), lambda i: (i, 0)),
            pl.BlockSpec((128, 128), lambda i: (0, 0)),
            pl.BlockSpec((BR, 128), lambda i: (i, 0)),
            pl.BlockSpec((BR, 128), lambda i: (i, 0)),
        ],
        out_specs=[
            pl.BlockSpec((BN, 128), lambda i: (i, 0)),
            pl.BlockSpec((BR, 128), lambda i: (i, 0)),
        ],
        out_shape=[
            jax.ShapeDtypeStruct((N, 128), jnp.float32),
            jax.ShapeDtypeStruct((R, 128), jnp.float32),
        ],
    )(x, Wcat, angp2, ewp2)


def _tc1(xy, p3, b1, Wbig):
    N = xy.shape[0]
    BN = N // 10

    def body(xy_ref, p0_ref, p1_ref, b_ref, w_ref,
             xf_ref, u_ref, v_ref, y2_ref):
        xs = xy_ref[...][:, :64]
        a = p0_ref[...][0, :, 64:] + p1_ref[...][0, :, 64:]
        x1 = jnp.maximum(xs + a + b_ref[...], 0.0)
        big = jnp.dot(x1, w_ref[...], preferred_element_type=jnp.float32)
        xf_ref[...] = big[:, :128]
        u_ref[...] = big[:, 128:256]
        v_ref[...] = big[:, 256:384]
        y2_ref[...] = big[:, 384:]

    blk = lambda i: (i, 0)
    full = lambda i: (0, 0)
    return pl.pallas_call(
        body,
        grid=(10,),
        in_specs=[
            pl.BlockSpec((BN, 128), blk),
            pl.BlockSpec((1, BN, 128), lambda i: (0, i, 0)),
            pl.BlockSpec((1, BN, 128), lambda i: (1, i, 0)),
            pl.BlockSpec((1, 64), full),
            pl.BlockSpec((64, 512), full),
        ],
        out_specs=[pl.BlockSpec((BN, 128), blk)] * 4,
        out_shape=[jax.ShapeDtypeStruct((N, 128), jnp.float32)] * 4,
    )(xy, p3, p3, b1, Wbig)


def _tc2(xf2, p3, b2, Wpq):
    N = xf2.shape[0]
    BN = N // 10

    def body(xf_ref, p0_ref, p1_ref, b_ref, w_ref, p_ref, q_ref):
        x2 = jnp.maximum(xf_ref[...] + p0_ref[...][0] + p1_ref[...][0]
                         + b_ref[...], 0.0)
        pq = jnp.dot(x2, w_ref[...], preferred_element_type=jnp.float32)
        p_ref[...] = pq[:, :128]
        q_ref[...] = pq[:, 128:]

    blk = lambda i: (i, 0)
    full = lambda i: (0, 0)
    return pl.pallas_call(
        body,
        grid=(10,),
        in_specs=[
            pl.BlockSpec((BN, 128), blk),
            pl.BlockSpec((1, BN, 128), lambda i: (0, i, 0)),
            pl.BlockSpec((1, BN, 128), lambda i: (1, i, 0)),
            pl.BlockSpec((1, 128), full),
            pl.BlockSpec((128, 256), full),
        ],
        out_specs=[pl.BlockSpec((BN, 128), blk)] * 2,
        out_shape=[jax.ShapeDtypeStruct((N, 128), jnp.float32)] * 2,
    )(xf2, p3, p3, b2, Wpq)


def _tc3(E, Ssum, ef, ewc, be1, We2c, be2, Wo1a, Wo1b, wo1c, bo1, Wo2, bo2):
    E2 = ef.shape[0]
    BE = 1600
    G = E2 // BE

    def body(e_ref, s_ref, ef_ref, ew_ref,
             b1_ref, w2c_ref, b2_ref,
             wo1a_ref, wo1b_ref, wo1c_ref, bo1_ref, wo2_ref, bo2_ref,
             out_ref):
        bf = jnp.bfloat16
        ew = ew_ref[...]
        e1 = jnp.maximum(ew * e_ref[...] + b1_ref[...], 0.0)
        e2 = jnp.maximum(
            ew * s_ref[...]
            + jnp.dot(e1.astype(bf), w2c_ref[...].astype(bf),
                      preferred_element_type=jnp.float32)
            + b2_ref[...], 0.0)
        h = (jnp.dot(e2.astype(bf), wo1a_ref[...].astype(bf),
                     preferred_element_type=jnp.float32)
             + jnp.dot(ef_ref[...].astype(bf), wo1b_ref[...].astype(bf),
                       preferred_element_type=jnp.float32)
             + ew * wo1c_ref[...]
             + bo1_ref[...])
        logits = jnp.dot(h.astype(bf), wo2_ref[...].astype(bf),
                         preferred_element_type=jnp.float32) + bo2_ref[...]
        m = jnp.max(logits, axis=-1, keepdims=True)
        ex = jnp.exp(logits - m)
        out_ref[...] = ex / jnp.sum(ex, axis=-1, keepdims=True)

    full = lambda i: (0, 0)
    blk = lambda i: (i, 0)
    return pl.pallas_call(
        body,
        grid=(G,),
        in_specs=[
            pl.BlockSpec((BE, 128), blk),
            pl.BlockSpec((BE, 128), blk),
            pl.BlockSpec((BE, 16), blk),
            pl.BlockSpec((BE, 1), blk),
            pl.BlockSpec((1, 128), full),
            pl.BlockSpec((128, 128), full),
            pl.BlockSpec((1, 128), full),
            pl.BlockSpec((128, 256), full),
            pl.BlockSpec((16, 256), full),
            pl.BlockSpec((1, 256), full),
            pl.BlockSpec((1, 256), full),
            pl.BlockSpec((256, 2), full),
            pl.BlockSpec((1, 2), full),
        ],
        out_specs=pl.BlockSpec((BE, 2), blk),
        out_shape=jax.ShapeDtypeStruct((E2, 2), jnp.float32),
    )(E, Ssum, ef, ewc, be1, We2c, be2, Wo1a, Wo1b, wo1c, bo1, Wo2, bo2)


def kernel(node_features, edge_features_1d, edge_index, angles, edge_weights,
           W_self1, W_nbr1, b_n1, We1, be1,
           W_self2, W_nbr2, b_n2, We2, be2,
           W_o1, b_o1, W_o2, b_o2):
    N = node_features.shape[0]
    E2 = angles.shape[0]
    CHUNK = 1024 * _NW
    E2P = ((E2 + CHUNK - 1) // CHUNK) * CHUNK
    pad = E2P - E2
    R = E2P // 128

    src = edge_index[0]
    dst = edge_index[1]
    ewf = jnp.concatenate([edge_weights, edge_weights])
    src2 = jnp.pad(src, (0, pad)).reshape(R, 128)
    dst2 = jnp.pad(dst, (0, pad)).reshape(R, 128)
    angp2 = jnp.pad(angles, (0, pad)).reshape(R, 128)
    ewp2 = jnp.pad(ewf, (0, pad)).reshape(R, 128)

    Wcat = jnp.concatenate([W_self1, W_nbr1], axis=1)
    Wbig = jnp.concatenate([W_self2, We1[:64], We1[64:], W_nbr2], axis=1)

    xy, coef2 = _tc0(node_features, Wcat, angp2, ewp2)
    agg1 = _sc_seg(xy, src2, dst2, coef2)
    xf2, U, V, Y2 = _tc1(xy, agg1.reshape(2, _NP, _D), b_n1.reshape(1, 64), Wbig)
    E = _sc_pair(U, V, src2, dst2)
    agg2 = _sc_seg(Y2, src2, dst2, coef2)
    Wpq = jnp.concatenate([We2[:128], We2[128:256]], axis=1)
    P, Q = _tc2(xf2, agg2.reshape(2, _NP, _D), b_n2.reshape(1, 128), Wpq)
    Ssum = _sc_pair(P, Q, src2, dst2)
    out = _tc3(E, Ssum, edge_features_1d, ewf.reshape(E2, 1),
               be1.reshape(1, 128),
               We2[256:], be2.reshape(1, 128),
               W_o1[:128], W_o1[128:144], W_o1[144:145], b_o1.reshape(1, 256),
               W_o2, b_o2.reshape(1, 2))
    return out
